# baseline (speedup 1.0000x reference)
import functools

import jax
import jax.numpy as jnp
from jax import lax
from jax.experimental import pallas as pl
from jax.experimental.pallas import tpu as pltpu
from jax.experimental.pallas import tpu_sc as plsc

NC = 2
NS = 16
NW = NC * NS
L = 16
CHUNK = 128


def _mm_body(x_ref, w_ref, o_ref):
    o_ref[...] = jnp.dot(x_ref[...], w_ref[...],
                         preferred_element_type=jnp.float32)


def _matmul(x, w):
    n, d_in = x.shape
    d_out = w.shape[1]
    bm = 2000
    grid = (n // bm,)
    return pl.pallas_call(
        _mm_body,
        grid=grid,
        in_specs=[
            pl.BlockSpec((bm, d_in), lambda i: (i, 0)),
            pl.BlockSpec((d_in, d_out), lambda i: (0, 0)),
        ],
        out_specs=pl.BlockSpec((bm, d_out), lambda i: (i, 0)),
        out_shape=jax.ShapeDtypeStruct((n, d_out), jnp.float32),
    )(x, w)


DH = 64


def _sc_agg_body(nch, n, xw_hbm, src_hbm, dst_hbm, w_hbm, z_hbm, part_hbm,
                 src_v, dst_v, w_v, r0, r1, acc_sh, gsems):
    c = lax.axis_index("c")
    s = lax.axis_index("s")
    rows = (r0, r1)

    def _issue_gather(j, b):
        pltpu.async_copy(xw_hbm.at[src_v.at[j]], rows[b], gsems.at[b])

    def _wait_gather(j, b):
        pltpu.make_async_copy(xw_hbm.at[src_v.at[j]], rows[b],
                              gsems.at[b]).wait()

    def _scale(j, b):
        buf = rows[b]
        joff = j * CHUNK

        def body(g, _):
            wvec = w_v[pl.ds(joff + g * L, L)]
            eb = g * L
            for i in range(L):
                wv = wvec[i]
                for cg in range(DH // L):
                    sl = pl.ds(cg * L, L)
                    buf[eb + i, sl] = buf[eb + i, sl] * wv
            return 0

        lax.fori_loop(0, CHUNK // L, body, 0)

    pltpu.sync_copy(src_hbm.at[c, s], src_v.at[pl.ds(0, nch)])
    pltpu.sync_copy(dst_hbm.at[s], dst_v)
    pltpu.sync_copy(w_hbm.at[s], w_v)

    zero_i = jnp.zeros((L,), jnp.int32)
    for r in range(2):
        for cg in range(CHUNK // L):
            src_v[nch + r, pl.ds(cg * L, L)] = zero_i

    _issue_gather(0, 0)
    _issue_gather(1, 1)

    rpt = (n // (8 * NS)) * 8
    tail = n - NS * rpt
    base = s * rpt

    pltpu.sync_copy(z_hbm.at[pl.ds(base, rpt)], acc_sh.at[pl.ds(base, rpt)])
    if tail:
        @pl.when(s == NS - 1)
        def _ztail():
            pltpu.sync_copy(z_hbm.at[pl.ds(NS * rpt, tail)],
                            acc_sh.at[pl.ds(NS * rpt, tail)])

    plsc.subcore_barrier()

    def _pair(jj, _):
        j0 = jj * 2
        j1 = j0 + 1
        _wait_gather(j0, 0)
        _scale(j0, 0)
        pltpu.sync_copy(r0, acc_sh.at[dst_v.at[j0]], add=True)
        _issue_gather(j0 + 2, 0)
        _wait_gather(j1, 1)
        _scale(j1, 1)
        pltpu.sync_copy(r1, acc_sh.at[dst_v.at[j1]], add=True)
        _issue_gather(j1 + 2, 1)
        return 0

    lax.fori_loop(0, nch // 2, _pair, 0)

    _wait_gather(nch, 0)
    _wait_gather(nch + 1, 1)

    plsc.subcore_barrier()

    pltpu.sync_copy(acc_sh.at[pl.ds(base, rpt)],
                    part_hbm.at[c, pl.ds(base, rpt)])
    if tail:
        @pl.when(s == NS - 1)
        def _wtail():
            pltpu.sync_copy(acc_sh.at[pl.ds(NS * rpt, tail)],
                            part_hbm.at[c, pl.ds(NS * rpt, tail)])


def _sc_aggregate(xw, src, dst, ew):
    n, d = xw.shape
    e = src.shape[0]
    nch = -(-e // (NS * CHUNK))
    if nch % 2:
        nch += 1
    e_pad = nch * NS * CHUNK
    pad = e_pad - e
    if pad:
        src = jnp.concatenate([src, jnp.zeros((pad,), jnp.int32)])
        dst = jnp.concatenate([dst, jnp.zeros((pad,), jnp.int32)])
        ew = jnp.concatenate([ew, jnp.zeros((pad,), jnp.float32)])
    src2 = jnp.stack([src * 2, src * 2 + 1]).reshape(NC, NS, nch, CHUNK)
    dst = dst.reshape(NS, nch, CHUNK)
    ew = ew.reshape(NS, nch * CHUNK)
    xw2 = xw.reshape(2 * n, DH)
    z = jnp.zeros((n, DH), jnp.float32)

    mesh = plsc.VectorSubcoreMesh(core_axis_name="c", subcore_axis_name="s")
    k = functools.partial(
        pl.kernel,
        mesh=mesh,
        compiler_params=pltpu.CompilerParams(use_tc_tiling_on_sc=False),
        out_type=jax.ShapeDtypeStruct((NC, n, DH), jnp.float32),
        scratch_types=[
            pltpu.VMEM((nch + 2, CHUNK), jnp.int32),
            pltpu.VMEM((nch, CHUNK), jnp.int32),
            pltpu.VMEM((nch * CHUNK,), jnp.float32),
            pltpu.VMEM((CHUNK, DH), jnp.float32),
            pltpu.VMEM((CHUNK, DH), jnp.float32),
            pltpu.VMEM_SHARED((n, DH), jnp.float32),
            pltpu.SemaphoreType.DMA((2,)),
        ],
    )(functools.partial(_sc_agg_body, nch, n))
    return k(xw2, src2, dst, ew, z)


def _fin_body(xw_ref, p_ref, skip_ref, bias_ref, o_ref):
    agg = jnp.concatenate([p_ref[0], p_ref[1]], axis=-1)
    v = xw_ref[...] * skip_ref[...] + agg + bias_ref[...]
    alpha = 1.6732632423543772848170429916717
    scale = 1.0507009873554804934193349852946
    o_ref[...] = scale * jnp.where(v > 0, v, alpha * (jnp.exp(v) - 1.0))


def _finalize(xw, parts, skip_weight, bias):
    n, d = xw.shape
    bm = 2000
    grid = (n // bm,)
    return pl.pallas_call(
        _fin_body,
        grid=grid,
        in_specs=[
            pl.BlockSpec((bm, d), lambda i: (i, 0)),
            pl.BlockSpec((NC, bm, DH), lambda i: (0, i, 0)),
            pl.BlockSpec((1, d), lambda i: (0, 0)),
            pl.BlockSpec((1, d), lambda i: (0, 0)),
        ],
        out_specs=pl.BlockSpec((bm, d), lambda i: (i, 0)),
        out_shape=jax.ShapeDtypeStruct((n, d), jnp.float32),
    )(xw, parts, skip_weight.reshape(1, d), bias.reshape(1, d))


def kernel(features, edge_index, edge_weight, kernel, bias, skip_weight):
    xw = _matmul(features, kernel)
    parts = _sc_aggregate(xw, edge_index[0], edge_index[1], edge_weight)
    return _finalize(xw, parts, skip_weight, bias)


# speedup vs baseline: 2.3637x; 2.3637x over previous
import functools

import jax
import jax.numpy as jnp
from jax import lax
from jax.experimental import pallas as pl
from jax.experimental.pallas import tpu as pltpu
from jax.experimental.pallas import tpu_sc as plsc

NC = 2
NS = 16
NW = NC * NS
L = 16
CHUNK = 128


def _mm_body(x_ref, w_ref, o_ref, p0_ref, p1_ref):
    xw = jnp.dot(x_ref[...], w_ref[...], preferred_element_type=jnp.float32)
    o_ref[...] = xw
    bits = lax.bitcast_convert_type(xw, jnp.int32)
    b = ((bits + 0x7FFF + ((bits >> 16) & 1)) >> 16) & 0xFFFF
    h = b.shape[1] // 2
    blo = b[:, :h]
    bhi = b[:, h:]
    p0_ref[...] = blo | (bhi << 16)
    p1_ref[...] = bhi | (blo << 16)


def _matmul(x, w):
    n, d_in = x.shape
    d_out = w.shape[1]
    bm = 2000
    grid = (n // bm,)
    return pl.pallas_call(
        _mm_body,
        grid=grid,
        in_specs=[
            pl.BlockSpec((bm, d_in), lambda i: (i, 0)),
            pl.BlockSpec((d_in, d_out), lambda i: (0, 0)),
        ],
        out_specs=[
            pl.BlockSpec((bm, d_out), lambda i: (i, 0)),
            pl.BlockSpec((bm, d_out // 2), lambda i: (i, 0)),
            pl.BlockSpec((bm, d_out // 2), lambda i: (i, 0)),
        ],
        out_shape=[
            jax.ShapeDtypeStruct((n, d_out), jnp.float32),
            jax.ShapeDtypeStruct((n, d_out // 2), jnp.int32),
            jax.ShapeDtypeStruct((n, d_out // 2), jnp.int32),
        ],
    )(x, w)


DH = 64
NSLOT = 6


def _sc_agg_body(nch, n, xwp0_hbm, xwp1_hbm, ep_hbm, z_hbm, part_hbm,
                 ering, gbuf, sbuf, xw_sh, acc_sh, dsem, esems):
    c = lax.axis_index("c")
    s = lax.axis_index("s")

    def _issue_edge(j, slot):
        pltpu.async_copy(ep_hbm.at[s, j], ering.at[slot], esems.at[slot])

    def _wait_edge(j, slot):
        pltpu.make_async_copy(ep_hbm.at[s, j], ering.at[slot],
                              esems.at[slot]).wait()

    for j in range(NSLOT - 1):
        _issue_edge(j, j)

    rpt = (n // (8 * NS)) * 8
    tail = n - NS * rpt
    base = s * rpt

    @pl.when(c == 0)
    def _stage0():
        pltpu.sync_copy(xwp0_hbm.at[pl.ds(base, rpt)],
                        xw_sh.at[pl.ds(base, rpt)])

    @pl.when(c == 1)
    def _stage1():
        pltpu.sync_copy(xwp1_hbm.at[pl.ds(base, rpt)],
                        xw_sh.at[pl.ds(base, rpt)])

    pltpu.sync_copy(z_hbm.at[pl.ds(base, rpt)], acc_sh.at[pl.ds(base, rpt)])
    if tail:
        @pl.when(s == NS - 1)
        def _ztail():
            pltpu.sync_copy(z_hbm.at[pl.ds(NS * rpt, tail)],
                            acc_sh.at[pl.ds(NS * rpt, tail)])

            @pl.when(c == 0)
            def _st0():
                pltpu.sync_copy(xwp0_hbm.at[pl.ds(NS * rpt, tail)],
                                xw_sh.at[pl.ds(NS * rpt, tail)])

            @pl.when(c == 1)
            def _st1():
                pltpu.sync_copy(xwp1_hbm.at[pl.ds(NS * rpt, tail)],
                                xw_sh.at[pl.ds(NS * rpt, tail)])

    plsc.subcore_barrier()

    def _process(slot):
        pltpu.sync_copy(xw_sh.at[ering.at[slot, 0]], gbuf)

        def _scale(g, _):
            wvec = lax.bitcast_convert_type(
                ering[slot, 2, pl.ds(g * L, L)], jnp.float32)
            eb = g * L
            for i in range(L):
                wv = wvec[i]
                for cg in range(DH // L):
                    sl = pl.ds(cg * L, L)
                    v = lax.bitcast_convert_type(
                        gbuf[eb + i, sl] << 16, jnp.float32)
                    sbuf[eb + i, sl] = v * wv
            return 0

        lax.fori_loop(0, CHUNK // L, _scale, 0)

        pltpu.sync_copy(sbuf, acc_sh.at[ering.at[slot, 1]], add=True)

    _wait_edge(0, 0)
    _process(0)
    _issue_edge(5, 5)

    def _group(gg, _):
        j0 = gg * NSLOT + 1
        for k in range(1, NSLOT + 1):
            j = j0 + k - 1
            _wait_edge(j, k % NSLOT)
            _process(k % NSLOT)
            _issue_edge(j + 5, (k - 1) % NSLOT)
        return 0

    lax.fori_loop(0, (nch - 1) // NSLOT, _group, 0)

    for j in range(nch, nch + 5):
        _wait_edge(j, j % NSLOT)

    plsc.subcore_barrier()

    pltpu.sync_copy(acc_sh.at[pl.ds(base, rpt)],
                    part_hbm.at[c, pl.ds(base, rpt)])
    if tail:
        @pl.when(s == NS - 1)
        def _wtail():
            pltpu.sync_copy(acc_sh.at[pl.ds(NS * rpt, tail)],
                            part_hbm.at[c, pl.ds(NS * rpt, tail)])


def _sc_aggregate(xwp0, xwp1, src, dst, ew):
    n = xwp0.shape[0]
    e = src.shape[0]
    nch = -(-e // (NS * CHUNK))
    while nch % NSLOT != 1:
        nch += 1
    e_pad = nch * NS * CHUNK
    pad = e_pad - e
    if pad:
        src = jnp.concatenate([src, jnp.zeros((pad,), jnp.int32)])
        dst = jnp.concatenate([dst, jnp.zeros((pad,), jnp.int32)])
        ew = jnp.concatenate([ew, jnp.zeros((pad,), jnp.float32)])
    wbits = lax.bitcast_convert_type(ew, jnp.int32)
    ep = jnp.stack([src.reshape(NS, nch, CHUNK),
                    dst.reshape(NS, nch, CHUNK),
                    wbits.reshape(NS, nch, CHUNK)], axis=2)
    ep = jnp.concatenate(
        [ep, jnp.zeros((NS, 5, 3, CHUNK), jnp.int32)], axis=1)
    z = jnp.zeros((n, DH), jnp.float32)

    mesh = plsc.VectorSubcoreMesh(core_axis_name="c", subcore_axis_name="s")
    k = functools.partial(
        pl.kernel,
        mesh=mesh,
        compiler_params=pltpu.CompilerParams(use_tc_tiling_on_sc=False),
        out_type=jax.ShapeDtypeStruct((NC, n, DH), jnp.float32),
        scratch_types=[
            pltpu.VMEM((NSLOT, 3, CHUNK), jnp.int32),
            pltpu.VMEM((CHUNK, DH), jnp.int32),
            pltpu.VMEM((CHUNK, DH), jnp.float32),
            pltpu.VMEM_SHARED((n, DH), jnp.int32),
            pltpu.VMEM_SHARED((n, DH), jnp.float32),
            pltpu.SemaphoreType.DMA,
            pltpu.SemaphoreType.DMA((NSLOT,)),
        ],
    )(functools.partial(_sc_agg_body, nch, n))
    return k(xwp0, xwp1, ep, z)


def _fin_body(xw_ref, p_ref, skip_ref, bias_ref, o_ref):
    agg = jnp.concatenate([p_ref[0], p_ref[1]], axis=-1)
    v = xw_ref[...] * skip_ref[...] + agg + bias_ref[...]
    alpha = 1.6732632423543772848170429916717
    scale = 1.0507009873554804934193349852946
    o_ref[...] = scale * jnp.where(v > 0, v, alpha * (jnp.exp(v) - 1.0))


def _finalize(xw, parts, skip_weight, bias):
    n, d = xw.shape
    bm = 2000
    grid = (n // bm,)
    return pl.pallas_call(
        _fin_body,
        grid=grid,
        in_specs=[
            pl.BlockSpec((bm, d), lambda i: (i, 0)),
            pl.BlockSpec((NC, bm, DH), lambda i: (0, i, 0)),
            pl.BlockSpec((1, d), lambda i: (0, 0)),
            pl.BlockSpec((1, d), lambda i: (0, 0)),
        ],
        out_specs=pl.BlockSpec((bm, d), lambda i: (i, 0)),
        out_shape=jax.ShapeDtypeStruct((n, d), jnp.float32),
    )(xw, parts, skip_weight.reshape(1, d), bias.reshape(1, d))


def kernel(features, edge_index, edge_weight, kernel, bias, skip_weight):
    xw, xwp0, xwp1 = _matmul(features, kernel)
    parts = _sc_aggregate(xwp0, xwp1,
                          edge_index[0], edge_index[1], edge_weight)
    return _finalize(xw, parts, skip_weight, bias)


# speedup vs baseline: 3.1158x; 1.3182x over previous
import functools

import jax
import jax.numpy as jnp
from jax import lax
from jax.experimental import pallas as pl
from jax.experimental.pallas import tpu as pltpu
from jax.experimental.pallas import tpu_sc as plsc

NC = 2
NS = 16
NW = NC * NS
L = 16
CHUNK = 128


def _mm_body(x_ref, w_ref, o_ref, p0_ref, p1_ref):
    xw = jnp.dot(x_ref[...], w_ref[...], preferred_element_type=jnp.float32)
    o_ref[...] = xw
    bits = lax.bitcast_convert_type(xw, jnp.int32)
    b = ((bits + 0x7FFF + ((bits >> 16) & 1)) >> 16) & 0xFFFF
    h = b.shape[1] // 2
    blo = b[:, :h]
    bhi = b[:, h:]
    p0_ref[...] = blo | (bhi << 16)
    p1_ref[...] = bhi | (blo << 16)


def _matmul(x, w):
    n, d_in = x.shape
    d_out = w.shape[1]
    bm = 2000
    grid = (n // bm,)
    return pl.pallas_call(
        _mm_body,
        grid=grid,
        in_specs=[
            pl.BlockSpec((bm, d_in), lambda i: (i, 0)),
            pl.BlockSpec((d_in, d_out), lambda i: (0, 0)),
        ],
        out_specs=[
            pl.BlockSpec((bm, d_out), lambda i: (i, 0)),
            pl.BlockSpec((bm, d_out // 2), lambda i: (i, 0)),
            pl.BlockSpec((bm, d_out // 2), lambda i: (i, 0)),
        ],
        out_shape=[
            jax.ShapeDtypeStruct((n, d_out), jnp.float32),
            jax.ShapeDtypeStruct((n, d_out // 2), jnp.int32),
            jax.ShapeDtypeStruct((n, d_out // 2), jnp.int32),
        ],
    )(x, w)


DH = 64
NSLOT = 6


def _sc_agg_body(nch, n, xwp0_hbm, xwp1_hbm, ep_hbm, z_hbm, part_hbm,
                 ering, g0, g1, sbuf, xw_sh, acc_sh, gsems, esems):
    c = lax.axis_index("c")
    s = lax.axis_index("s")
    gbufs = (g0, g1)

    def _issue_edge(j, slot):
        pltpu.async_copy(ep_hbm.at[s, j], ering.at[slot], esems.at[slot])

    def _wait_edge(j, slot):
        pltpu.make_async_copy(ep_hbm.at[s, j], ering.at[slot],
                              esems.at[slot]).wait()

    def _issue_gather(slot, b):
        pltpu.async_copy(xw_sh.at[ering.at[slot, 0]], gbufs[b], gsems.at[b])

    def _wait_gather(slot, b):
        pltpu.make_async_copy(xw_sh.at[ering.at[slot, 0]], gbufs[b],
                              gsems.at[b]).wait()

    for j in range(NSLOT - 1):
        _issue_edge(j, j)

    rpt = (n // (8 * NS)) * 8
    tail = n - NS * rpt
    base = s * rpt

    @pl.when(c == 0)
    def _stage0():
        pltpu.sync_copy(xwp0_hbm.at[pl.ds(base, rpt)],
                        xw_sh.at[pl.ds(base, rpt)])

    @pl.when(c == 1)
    def _stage1():
        pltpu.sync_copy(xwp1_hbm.at[pl.ds(base, rpt)],
                        xw_sh.at[pl.ds(base, rpt)])

    pltpu.sync_copy(z_hbm.at[pl.ds(base, rpt)], acc_sh.at[pl.ds(base, rpt)])
    if tail:
        @pl.when(s == NS - 1)
        def _ztail():
            pltpu.sync_copy(z_hbm.at[pl.ds(NS * rpt, tail)],
                            acc_sh.at[pl.ds(NS * rpt, tail)])

            @pl.when(c == 0)
            def _st0():
                pltpu.sync_copy(xwp0_hbm.at[pl.ds(NS * rpt, tail)],
                                xw_sh.at[pl.ds(NS * rpt, tail)])

            @pl.when(c == 1)
            def _st1():
                pltpu.sync_copy(xwp1_hbm.at[pl.ds(NS * rpt, tail)],
                                xw_sh.at[pl.ds(NS * rpt, tail)])

    plsc.subcore_barrier()

    def _process(slot, b):
        gbuf = gbufs[b]

        def _scale(g, _):
            wvec = lax.bitcast_convert_type(
                ering[slot, 2, pl.ds(g * L, L)], jnp.float32)
            eb = g * L
            for i in range(L):
                wv = wvec[i]
                for cg in range(DH // L):
                    sl = pl.ds(cg * L, L)
                    v = lax.bitcast_convert_type(
                        gbuf[eb + i, sl] << 16, jnp.float32)
                    sbuf[eb + i, sl] = v * wv
            return 0

        lax.fori_loop(0, CHUNK // L, _scale, 0)

        pltpu.sync_copy(sbuf, acc_sh.at[ering.at[slot, 1]], add=True)

    _wait_edge(0, 0)
    _issue_gather(0, 0)
    _wait_edge(1, 1)
    _issue_gather(1, 1)
    _wait_gather(0, 0)
    _process(0, 0)
    _issue_edge(5, 5)

    def _group(gg, _):
        j0 = gg * NSLOT + 1
        for k in range(1, NSLOT + 1):
            j = j0 + k - 1
            _wait_edge(j + 1, (k + 1) % NSLOT)
            _issue_gather((k + 1) % NSLOT, (k + 1) % 2)
            _wait_gather(k % NSLOT, k % 2)
            _process(k % NSLOT, k % 2)
            _issue_edge(j + 5, (k - 1) % NSLOT)
        return 0

    lax.fori_loop(0, (nch - 1) // NSLOT, _group, 0)

    _wait_gather(nch % NSLOT, nch % 2)
    for j in range(nch + 1, nch + 5):
        _wait_edge(j, j % NSLOT)

    plsc.subcore_barrier()

    pltpu.sync_copy(acc_sh.at[pl.ds(base, rpt)],
                    part_hbm.at[c, pl.ds(base, rpt)])
    if tail:
        @pl.when(s == NS - 1)
        def _wtail():
            pltpu.sync_copy(acc_sh.at[pl.ds(NS * rpt, tail)],
                            part_hbm.at[c, pl.ds(NS * rpt, tail)])


def _sc_aggregate(xwp0, xwp1, src, dst, ew):
    n = xwp0.shape[0]
    e = src.shape[0]
    nch = -(-e // (NS * CHUNK))
    while nch % NSLOT != 1:
        nch += 1
    e_pad = nch * NS * CHUNK
    pad = e_pad - e
    if pad:
        src = jnp.concatenate([src, jnp.zeros((pad,), jnp.int32)])
        dst = jnp.concatenate([dst, jnp.zeros((pad,), jnp.int32)])
        ew = jnp.concatenate([ew, jnp.zeros((pad,), jnp.float32)])
    wbits = lax.bitcast_convert_type(ew, jnp.int32)
    ep = jnp.stack([src.reshape(NS, nch, CHUNK),
                    dst.reshape(NS, nch, CHUNK),
                    wbits.reshape(NS, nch, CHUNK)], axis=2)
    ep = jnp.concatenate(
        [ep, jnp.zeros((NS, 5, 3, CHUNK), jnp.int32)], axis=1)
    z = jnp.zeros((n, DH), jnp.float32)

    mesh = plsc.VectorSubcoreMesh(core_axis_name="c", subcore_axis_name="s")
    k = functools.partial(
        pl.kernel,
        mesh=mesh,
        compiler_params=pltpu.CompilerParams(use_tc_tiling_on_sc=False),
        out_type=jax.ShapeDtypeStruct((NC, n, DH), jnp.float32),
        scratch_types=[
            pltpu.VMEM((NSLOT, 3, CHUNK), jnp.int32),
            pltpu.VMEM((CHUNK, DH), jnp.int32),
            pltpu.VMEM((CHUNK, DH), jnp.int32),
            pltpu.VMEM((CHUNK, DH), jnp.float32),
            pltpu.VMEM_SHARED((n, DH), jnp.int32),
            pltpu.VMEM_SHARED((n, DH), jnp.float32),
            pltpu.SemaphoreType.DMA((2,)),
            pltpu.SemaphoreType.DMA((NSLOT,)),
        ],
    )(functools.partial(_sc_agg_body, nch, n))
    return k(xwp0, xwp1, ep, z)


def _fin_body(xw_ref, p_ref, skip_ref, bias_ref, o_ref):
    agg = jnp.concatenate([p_ref[0], p_ref[1]], axis=-1)
    v = xw_ref[...] * skip_ref[...] + agg + bias_ref[...]
    alpha = 1.6732632423543772848170429916717
    scale = 1.0507009873554804934193349852946
    o_ref[...] = scale * jnp.where(v > 0, v, alpha * (jnp.exp(v) - 1.0))


def _finalize(xw, parts, skip_weight, bias):
    n, d = xw.shape
    bm = 2000
    grid = (n // bm,)
    return pl.pallas_call(
        _fin_body,
        grid=grid,
        in_specs=[
            pl.BlockSpec((bm, d), lambda i: (i, 0)),
            pl.BlockSpec((NC, bm, DH), lambda i: (0, i, 0)),
            pl.BlockSpec((1, d), lambda i: (0, 0)),
            pl.BlockSpec((1, d), lambda i: (0, 0)),
        ],
        out_specs=pl.BlockSpec((bm, d), lambda i: (i, 0)),
        out_shape=jax.ShapeDtypeStruct((n, d), jnp.float32),
    )(xw, parts, skip_weight.reshape(1, d), bias.reshape(1, d))


def kernel(features, edge_index, edge_weight, kernel, bias, skip_weight):
    xw, xwp0, xwp1 = _matmul(features, kernel)
    parts = _sc_aggregate(xwp0, xwp1,
                          edge_index[0], edge_index[1], edge_weight)
    return _finalize(xw, parts, skip_weight, bias)


# speedup vs baseline: 3.2179x; 1.0328x over previous
import functools

import jax
import jax.numpy as jnp
from jax import lax
from jax.experimental import pallas as pl
from jax.experimental.pallas import tpu as pltpu
from jax.experimental.pallas import tpu_sc as plsc

NC = 2
NS = 16
NW = NC * NS
L = 16
CHUNK = 128


def _mm_body(x_ref, w_ref, o_ref, p0_ref, p1_ref):
    xw = jnp.dot(x_ref[...], w_ref[...], preferred_element_type=jnp.float32)
    o_ref[...] = xw
    bits = lax.bitcast_convert_type(xw, jnp.int32)
    b = ((bits + 0x7FFF + ((bits >> 16) & 1)) >> 16) & 0xFFFF
    h = b.shape[1] // 2
    blo = b[:, :h]
    bhi = b[:, h:]
    p0_ref[...] = blo | (bhi << 16)
    p1_ref[...] = bhi | (blo << 16)


def _matmul(x, w):
    n, d_in = x.shape
    d_out = w.shape[1]
    bm = 2000
    grid = (n // bm,)
    return pl.pallas_call(
        _mm_body,
        grid=grid,
        in_specs=[
            pl.BlockSpec((bm, d_in), lambda i: (i, 0)),
            pl.BlockSpec((d_in, d_out), lambda i: (0, 0)),
        ],
        out_specs=[
            pl.BlockSpec((bm, d_out), lambda i: (i, 0)),
            pl.BlockSpec((bm, d_out // 2), lambda i: (i, 0)),
            pl.BlockSpec((bm, d_out // 2), lambda i: (i, 0)),
        ],
        out_shape=[
            jax.ShapeDtypeStruct((n, d_out), jnp.float32),
            jax.ShapeDtypeStruct((n, d_out // 2), jnp.int32),
            jax.ShapeDtypeStruct((n, d_out // 2), jnp.int32),
        ],
    )(x, w)


DH = 64
NSLOT = 6


def _sc_agg_body(nch, n, xwp0_hbm, xwp1_hbm, ep_hbm, z_hbm, part_hbm,
                 ering, g0, g1, s0, s1, xw_sh, acc_sh, gsems, ssems, esems):
    c = lax.axis_index("c")
    s = lax.axis_index("s")
    gbufs = (g0, g1)
    sbufs = (s0, s1)

    def _issue_edge(j, slot):
        pltpu.async_copy(ep_hbm.at[s, j], ering.at[slot], esems.at[slot])

    def _wait_edge(j, slot):
        pltpu.make_async_copy(ep_hbm.at[s, j], ering.at[slot],
                              esems.at[slot]).wait()

    def _issue_gather(slot, b):
        pltpu.async_copy(xw_sh.at[ering.at[slot, 0]], gbufs[b], gsems.at[b])

    def _wait_gather(slot, b):
        pltpu.make_async_copy(xw_sh.at[ering.at[slot, 0]], gbufs[b],
                              gsems.at[b]).wait()

    def _issue_scatter(slot, b):
        pltpu.async_copy(sbufs[b], acc_sh.at[ering.at[slot, 1]],
                         ssems.at[b], add=True)

    def _wait_scatter(slot, b):
        pltpu.make_async_copy(sbufs[b], acc_sh.at[ering.at[slot, 1]],
                              ssems.at[b]).wait()

    for j in range(4):
        _issue_edge(j, j)

    rpt = (n // (8 * NS)) * 8
    tail = n - NS * rpt
    base = s * rpt

    @pl.when(c == 0)
    def _stage0():
        pltpu.sync_copy(xwp0_hbm.at[pl.ds(base, rpt)],
                        xw_sh.at[pl.ds(base, rpt)])

    @pl.when(c == 1)
    def _stage1():
        pltpu.sync_copy(xwp1_hbm.at[pl.ds(base, rpt)],
                        xw_sh.at[pl.ds(base, rpt)])

    pltpu.sync_copy(z_hbm.at[pl.ds(base, rpt)], acc_sh.at[pl.ds(base, rpt)])
    if tail:
        @pl.when(s == NS - 1)
        def _ztail():
            pltpu.sync_copy(z_hbm.at[pl.ds(NS * rpt, tail)],
                            acc_sh.at[pl.ds(NS * rpt, tail)])

            @pl.when(c == 0)
            def _st0():
                pltpu.sync_copy(xwp0_hbm.at[pl.ds(NS * rpt, tail)],
                                xw_sh.at[pl.ds(NS * rpt, tail)])

            @pl.when(c == 1)
            def _st1():
                pltpu.sync_copy(xwp1_hbm.at[pl.ds(NS * rpt, tail)],
                                xw_sh.at[pl.ds(NS * rpt, tail)])

    plsc.subcore_barrier()

    def _scale(slot, b):
        gbuf = gbufs[b]
        sbuf = sbufs[b]

        def body(g, _):
            wvec = lax.bitcast_convert_type(
                ering[slot, 2, pl.ds(g * L, L)], jnp.float32)
            eb = g * L
            for i in range(L):
                wv = wvec[i]
                for cg in range(DH // L):
                    sl = pl.ds(cg * L, L)
                    v = lax.bitcast_convert_type(
                        gbuf[eb + i, sl] << 16, jnp.float32)
                    sbuf[eb + i, sl] = v * wv
            return 0

        lax.fori_loop(0, CHUNK // L, body, 0)

    _wait_edge(0, 0)
    _issue_gather(0, 0)
    _wait_edge(1, 1)
    _issue_gather(1, 1)

    _wait_gather(0, 0)
    _scale(0, 0)
    _issue_scatter(0, 0)
    _issue_edge(4, 4)

    _wait_edge(2, 2)
    _issue_gather(2, 0)
    _wait_gather(1, 1)
    _scale(1, 1)
    _issue_scatter(1, 1)
    _issue_edge(5, 5)

    def _group(gg, _):
        j0 = gg * NSLOT + 2
        for i in range(NSLOT):
            j = j0 + i
            k = (2 + i) % NSLOT
            b = (2 + i) % 2
            _wait_edge(j + 1, (k + 1) % NSLOT)
            _issue_gather((k + 1) % NSLOT, (b + 1) % 2)
            _wait_gather(k, b)
            _wait_scatter((k - 2) % NSLOT, b)
            _scale(k, b)
            _issue_scatter(k, b)
            _issue_edge(j + 4, (k - 2) % NSLOT)
        return 0

    lax.fori_loop(0, (nch - 2) // NSLOT, _group, 0)

    _wait_gather(nch % NSLOT, nch % 2)
    _wait_scatter((nch - 2) % NSLOT, (nch - 2) % 2)
    _wait_scatter((nch - 1) % NSLOT, (nch - 1) % 2)
    for j in range(nch + 1, nch + 4):
        _wait_edge(j, j % NSLOT)

    plsc.subcore_barrier()

    pltpu.sync_copy(acc_sh.at[pl.ds(base, rpt)],
                    part_hbm.at[c, pl.ds(base, rpt)])
    if tail:
        @pl.when(s == NS - 1)
        def _wtail():
            pltpu.sync_copy(acc_sh.at[pl.ds(NS * rpt, tail)],
                            part_hbm.at[c, pl.ds(NS * rpt, tail)])


def _sc_aggregate(xwp0, xwp1, src, dst, ew):
    n = xwp0.shape[0]
    e = src.shape[0]
    nch = -(-e // (NS * CHUNK))
    while nch % NSLOT != 2:
        nch += 1
    e_pad = nch * NS * CHUNK
    pad = e_pad - e
    if pad:
        src = jnp.concatenate([src, jnp.zeros((pad,), jnp.int32)])
        dst = jnp.concatenate([dst, jnp.zeros((pad,), jnp.int32)])
        ew = jnp.concatenate([ew, jnp.zeros((pad,), jnp.float32)])
    wbits = lax.bitcast_convert_type(ew, jnp.int32)
    ep = jnp.stack([src.reshape(NS, nch, CHUNK),
                    dst.reshape(NS, nch, CHUNK),
                    wbits.reshape(NS, nch, CHUNK)], axis=2)
    ep = jnp.concatenate(
        [ep, jnp.zeros((NS, 5, 3, CHUNK), jnp.int32)], axis=1)
    z = jnp.zeros((n, DH), jnp.float32)

    mesh = plsc.VectorSubcoreMesh(core_axis_name="c", subcore_axis_name="s")
    k = functools.partial(
        pl.kernel,
        mesh=mesh,
        compiler_params=pltpu.CompilerParams(use_tc_tiling_on_sc=False),
        out_type=jax.ShapeDtypeStruct((NC, n, DH), jnp.float32),
        scratch_types=[
            pltpu.VMEM((NSLOT, 3, CHUNK), jnp.int32),
            pltpu.VMEM((CHUNK, DH), jnp.int32),
            pltpu.VMEM((CHUNK, DH), jnp.int32),
            pltpu.VMEM((CHUNK, DH), jnp.float32),
            pltpu.VMEM((CHUNK, DH), jnp.float32),
            pltpu.VMEM_SHARED((n, DH), jnp.int32),
            pltpu.VMEM_SHARED((n, DH), jnp.float32),
            pltpu.SemaphoreType.DMA((2,)),
            pltpu.SemaphoreType.DMA((2,)),
            pltpu.SemaphoreType.DMA((NSLOT,)),
        ],
    )(functools.partial(_sc_agg_body, nch, n))
    return k(xwp0, xwp1, ep, z)


def _fin_body(xw_ref, p_ref, skip_ref, bias_ref, o_ref):
    agg = jnp.concatenate([p_ref[0], p_ref[1]], axis=-1)
    v = xw_ref[...] * skip_ref[...] + agg + bias_ref[...]
    alpha = 1.6732632423543772848170429916717
    scale = 1.0507009873554804934193349852946
    o_ref[...] = scale * jnp.where(v > 0, v, alpha * (jnp.exp(v) - 1.0))


def _finalize(xw, parts, skip_weight, bias):
    n, d = xw.shape
    bm = 2000
    grid = (n // bm,)
    return pl.pallas_call(
        _fin_body,
        grid=grid,
        in_specs=[
            pl.BlockSpec((bm, d), lambda i: (i, 0)),
            pl.BlockSpec((NC, bm, DH), lambda i: (0, i, 0)),
            pl.BlockSpec((1, d), lambda i: (0, 0)),
            pl.BlockSpec((1, d), lambda i: (0, 0)),
        ],
        out_specs=pl.BlockSpec((bm, d), lambda i: (i, 0)),
        out_shape=jax.ShapeDtypeStruct((n, d), jnp.float32),
    )(xw, parts, skip_weight.reshape(1, d), bias.reshape(1, d))


def kernel(features, edge_index, edge_weight, kernel, bias, skip_weight):
    xw, xwp0, xwp1 = _matmul(features, kernel)
    parts = _sc_aggregate(xwp0, xwp1,
                          edge_index[0], edge_index[1], edge_weight)
    return _finalize(xw, parts, skip_weight, bias)


# speedup vs baseline: 3.3009x; 1.0258x over previous
import functools

import jax
import jax.numpy as jnp
from jax import lax
from jax.experimental import pallas as pl
from jax.experimental.pallas import tpu as pltpu
from jax.experimental.pallas import tpu_sc as plsc

NC = 2
NS = 16
NW = NC * NS
L = 16
CHUNK = 128


def _mm_body(x_ref, w_ref, o_ref, p0_ref, p1_ref):
    xw = jnp.dot(x_ref[...], w_ref[...], preferred_element_type=jnp.float32)
    o_ref[...] = xw
    bits = lax.bitcast_convert_type(xw, jnp.int32)
    b = ((bits + 0x7FFF + ((bits >> 16) & 1)) >> 16) & 0xFFFF
    h = b.shape[1] // 2
    blo = b[:, :h]
    bhi = b[:, h:]
    p0_ref[...] = blo | (bhi << 16)
    p1_ref[...] = bhi | (blo << 16)


def _matmul(x, w):
    n, d_in = x.shape
    d_out = w.shape[1]
    bm = 2000
    grid = (n // bm,)
    return pl.pallas_call(
        _mm_body,
        grid=grid,
        in_specs=[
            pl.BlockSpec((bm, d_in), lambda i: (i, 0)),
            pl.BlockSpec((d_in, d_out), lambda i: (0, 0)),
        ],
        out_specs=[
            pl.BlockSpec((bm, d_out), lambda i: (i, 0)),
            pl.BlockSpec((bm, d_out // 2), lambda i: (i, 0)),
            pl.BlockSpec((bm, d_out // 2), lambda i: (i, 0)),
        ],
        out_shape=[
            jax.ShapeDtypeStruct((n, d_out), jnp.float32),
            jax.ShapeDtypeStruct((n, d_out // 2), jnp.int32),
            jax.ShapeDtypeStruct((n, d_out // 2), jnp.int32),
        ],
    )(x, w)


DH = 64
NSLOT = 6


def _sc_agg_body(nch, n, xwp0_hbm, xwp1_hbm, src_hbm, dst_hbm, w_hbm,
                 z_hbm, part_hbm,
                 sring, dring, wring, g0, g1, s0, s1, xw_sh, acc_sh,
                 gsems, ssems, s_sems, d_sems, w_sems):
    c = lax.axis_index("c")
    s = lax.axis_index("s")
    gbufs = (g0, g1)
    sbufs = (s0, s1)

    def _issue_edge(j, slot):
        off = (s * nch + j) * CHUNK
        pltpu.async_copy(src_hbm.at[pl.ds(off, CHUNK)], sring.at[slot],
                         s_sems.at[slot])
        pltpu.async_copy(dst_hbm.at[pl.ds(off, CHUNK)], dring.at[slot],
                         d_sems.at[slot])
        pltpu.async_copy(w_hbm.at[pl.ds(off, CHUNK)], wring.at[slot],
                         w_sems.at[slot])

    def _wait_edge(j, slot):
        off = (s * nch + j) * CHUNK
        pltpu.make_async_copy(src_hbm.at[pl.ds(off, CHUNK)], sring.at[slot],
                              s_sems.at[slot]).wait()
        pltpu.make_async_copy(dst_hbm.at[pl.ds(off, CHUNK)], dring.at[slot],
                              d_sems.at[slot]).wait()
        pltpu.make_async_copy(w_hbm.at[pl.ds(off, CHUNK)], wring.at[slot],
                              w_sems.at[slot]).wait()

    def _issue_gather(slot, b):
        pltpu.async_copy(xw_sh.at[sring.at[slot]], gbufs[b], gsems.at[b])

    def _wait_gather(slot, b):
        pltpu.make_async_copy(xw_sh.at[sring.at[slot]], gbufs[b],
                              gsems.at[b]).wait()

    def _issue_scatter(slot, b):
        pltpu.async_copy(sbufs[b], acc_sh.at[dring.at[slot]],
                         ssems.at[b], add=True)

    def _wait_scatter(slot, b):
        pltpu.make_async_copy(sbufs[b], acc_sh.at[dring.at[slot]],
                              ssems.at[b]).wait()

    for j in range(4):
        _issue_edge(j, j)

    rpt = (n // (8 * NS)) * 8
    tail = n - NS * rpt
    base = s * rpt

    @pl.when(c == 0)
    def _stage0():
        pltpu.sync_copy(xwp0_hbm.at[pl.ds(base, rpt)],
                        xw_sh.at[pl.ds(base, rpt)])

    @pl.when(c == 1)
    def _stage1():
        pltpu.sync_copy(xwp1_hbm.at[pl.ds(base, rpt)],
                        xw_sh.at[pl.ds(base, rpt)])

    pltpu.sync_copy(z_hbm.at[pl.ds(base, rpt)], acc_sh.at[pl.ds(base, rpt)])
    if tail:
        @pl.when(s == NS - 1)
        def _ztail():
            pltpu.sync_copy(z_hbm.at[pl.ds(NS * rpt, tail)],
                            acc_sh.at[pl.ds(NS * rpt, tail)])

            @pl.when(c == 0)
            def _st0():
                pltpu.sync_copy(xwp0_hbm.at[pl.ds(NS * rpt, tail)],
                                xw_sh.at[pl.ds(NS * rpt, tail)])

            @pl.when(c == 1)
            def _st1():
                pltpu.sync_copy(xwp1_hbm.at[pl.ds(NS * rpt, tail)],
                                xw_sh.at[pl.ds(NS * rpt, tail)])

    plsc.subcore_barrier()

    def _scale(slot, b):
        gbuf = gbufs[b]
        sbuf = sbufs[b]

        def body(g, _):
            wvec = wring[slot, pl.ds(g * L, L)]
            eb = g * L
            for i in range(L):
                wv = wvec[i]
                for cg in range(DH // L):
                    sl = pl.ds(cg * L, L)
                    v = lax.bitcast_convert_type(
                        gbuf[eb + i, sl] << 16, jnp.float32)
                    sbuf[eb + i, sl] = v * wv
            return 0

        lax.fori_loop(0, CHUNK // L, body, 0)

    _wait_edge(0, 0)
    _issue_gather(0, 0)
    _wait_edge(1, 1)
    _issue_gather(1, 1)

    _wait_gather(0, 0)
    _scale(0, 0)
    _issue_scatter(0, 0)
    _issue_edge(4, 4)

    _wait_edge(2, 2)
    _issue_gather(2, 0)
    _wait_gather(1, 1)
    _scale(1, 1)
    _issue_scatter(1, 1)
    _issue_edge(5, 5)

    def _group(gg, _):
        j0 = gg * NSLOT + 2
        for i in range(NSLOT):
            j = j0 + i
            k = (2 + i) % NSLOT
            b = (2 + i) % 2
            _wait_edge(j + 1, (k + 1) % NSLOT)
            _issue_gather((k + 1) % NSLOT, (b + 1) % 2)
            _wait_gather(k, b)
            _wait_scatter((k - 2) % NSLOT, b)
            _scale(k, b)
            _issue_scatter(k, b)
            _issue_edge(j + 4, (k - 2) % NSLOT)
        return 0

    lax.fori_loop(0, (nch - 2) // NSLOT, _group, 0)

    _wait_gather(nch % NSLOT, nch % 2)
    _wait_scatter((nch - 2) % NSLOT, (nch - 2) % 2)
    _wait_scatter((nch - 1) % NSLOT, (nch - 1) % 2)
    for j in range(nch + 1, nch + 4):
        _wait_edge(j, j % NSLOT)

    plsc.subcore_barrier()

    pltpu.sync_copy(acc_sh.at[pl.ds(base, rpt)],
                    part_hbm.at[c, pl.ds(base, rpt)])
    if tail:
        @pl.when(s == NS - 1)
        def _wtail():
            pltpu.sync_copy(acc_sh.at[pl.ds(NS * rpt, tail)],
                            part_hbm.at[c, pl.ds(NS * rpt, tail)])


def _sc_aggregate(xwp0, xwp1, src, dst, ew):
    n = xwp0.shape[0]
    e = src.shape[0]
    nch = -(-e // (NS * CHUNK))
    while nch % NSLOT != 2:
        nch += 1
    e_pad = nch * NS * CHUNK
    pad = e_pad - e + 4 * CHUNK
    src = jnp.concatenate([src, jnp.zeros((pad,), jnp.int32)])
    dst = jnp.concatenate([dst, jnp.zeros((pad,), jnp.int32)])
    ew = jnp.concatenate([ew, jnp.zeros((pad,), jnp.float32)])
    z = jnp.zeros((n, DH), jnp.float32)

    mesh = plsc.VectorSubcoreMesh(core_axis_name="c", subcore_axis_name="s")
    k = functools.partial(
        pl.kernel,
        mesh=mesh,
        compiler_params=pltpu.CompilerParams(use_tc_tiling_on_sc=False),
        out_type=jax.ShapeDtypeStruct((NC, n, DH), jnp.float32),
        scratch_types=[
            pltpu.VMEM((NSLOT, CHUNK), jnp.int32),
            pltpu.VMEM((NSLOT, CHUNK), jnp.int32),
            pltpu.VMEM((NSLOT, CHUNK), jnp.float32),
            pltpu.VMEM((CHUNK, DH), jnp.int32),
            pltpu.VMEM((CHUNK, DH), jnp.int32),
            pltpu.VMEM((CHUNK, DH), jnp.float32),
            pltpu.VMEM((CHUNK, DH), jnp.float32),
            pltpu.VMEM_SHARED((n, DH), jnp.int32),
            pltpu.VMEM_SHARED((n, DH), jnp.float32),
            pltpu.SemaphoreType.DMA((2,)),
            pltpu.SemaphoreType.DMA((2,)),
            pltpu.SemaphoreType.DMA((NSLOT,)),
            pltpu.SemaphoreType.DMA((NSLOT,)),
            pltpu.SemaphoreType.DMA((NSLOT,)),
        ],
    )(functools.partial(_sc_agg_body, nch, n))
    return k(xwp0, xwp1, src, dst, ew, z)


def _fin_body(xw_ref, p_ref, skip_ref, bias_ref, o_ref):
    agg = jnp.concatenate([p_ref[0], p_ref[1]], axis=-1)
    v = xw_ref[...] * skip_ref[...] + agg + bias_ref[...]
    alpha = 1.6732632423543772848170429916717
    scale = 1.0507009873554804934193349852946
    o_ref[...] = scale * jnp.where(v > 0, v, alpha * (jnp.exp(v) - 1.0))


def _finalize(xw, parts, skip_weight, bias):
    n, d = xw.shape
    bm = 2000
    grid = (n // bm,)
    return pl.pallas_call(
        _fin_body,
        grid=grid,
        in_specs=[
            pl.BlockSpec((bm, d), lambda i: (i, 0)),
            pl.BlockSpec((NC, bm, DH), lambda i: (0, i, 0)),
            pl.BlockSpec((1, d), lambda i: (0, 0)),
            pl.BlockSpec((1, d), lambda i: (0, 0)),
        ],
        out_specs=pl.BlockSpec((bm, d), lambda i: (i, 0)),
        out_shape=jax.ShapeDtypeStruct((n, d), jnp.float32),
    )(xw, parts, skip_weight.reshape(1, d), bias.reshape(1, d))


def kernel(features, edge_index, edge_weight, kernel, bias, skip_weight):
    xw, xwp0, xwp1 = _matmul(features, kernel)
    parts = _sc_aggregate(xwp0, xwp1,
                          edge_index[0], edge_index[1], edge_weight)
    return _finalize(xw, parts, skip_weight, bias)
